# Initial kernel scaffold; baseline (speedup 1.0000x reference)
import jax
import jax.numpy as jnp
from jax.experimental import pallas as pl


def kernel(h, batch, node_mask, paratope_prob, sasa_prior, W):
    raise NotImplementedError("write your pallas kernel here")


# speedup vs baseline: 5.3990x; 5.3990x over previous
import functools

import jax
import jax.numpy as jnp
from jax.experimental import pallas as pl
from jax.experimental.pallas import tpu as pltpu

_BETA = 1.0
_GAMMA = 0.5
_NSEG = 512
_NEG = -1e30


def _body(h_ref, b_ref, mk_ref, pp_ref, ss_ref, w_ref, out_ref, m_ref, s_ref):
    p = pl.program_id(0)
    i = pl.program_id(1)
    r = h_ref.shape[0]

    h = h_ref[...]
    lcol = jnp.dot(h, w_ref[...], preferred_element_type=jnp.float32)
    lrow = lcol.reshape(1, r)
    batch = b_ref[0]
    mask = mk_ref[0] > 0.5
    ml = jnp.where(mask, lrow + _BETA * pp_ref[0] + _GAMMA * ss_ref[0], _NEG)

    iot = jax.lax.broadcasted_iota(jnp.int32, (_NSEG, r), 0)
    oh = iot == batch
    ohf = oh.astype(jnp.float32)

    @pl.when((p == 0) & (i == 0))
    def _init_stats():
        m_ref[...] = jnp.full((1, _NSEG), _NEG, jnp.float32)
        s_ref[...] = jnp.zeros((1, _NSEG), jnp.float32)

    @pl.when(p == 0)
    def _pass_stats():
        bmax = jnp.max(jnp.where(oh, ml, _NEG), axis=1).reshape(1, _NSEG)
        m_old = m_ref[...]
        m_new = jnp.maximum(m_old, bmax)
        rowm = jnp.dot(m_new, ohf, preferred_element_type=jnp.float32)
        e = jnp.where(mask, jnp.exp(ml - rowm), 0.0)
        bsum = jnp.sum(jnp.where(oh, e, 0.0), axis=1).reshape(1, _NSEG)
        s_ref[...] = s_ref[...] * jnp.exp(m_old - m_new) + bsum
        m_ref[...] = m_new

    @pl.when(p == 1)
    def _pass_accum():
        @pl.when(i == 0)
        def _init_out():
            out_ref[...] = jnp.zeros_like(out_ref)

        m = m_ref[...]
        m_safe = jnp.where(m > -1e29, m, 0.0)
        d = s_ref[...]
        rowm = jnp.dot(m_safe, ohf, preferred_element_type=jnp.float32)
        rowd = jnp.dot(d, ohf, preferred_element_type=jnp.float32)
        e = jnp.where(mask, jnp.exp(ml - rowm), 0.0)
        w = e / jnp.where(rowd > 0, rowd, 1.0)
        wh = w.reshape(r, 1) * h
        out_ref[...] += jnp.dot(ohf, wh, preferred_element_type=jnp.float32)


def _pick_block(n):
    for r in range(2048, 7, -8):
        if n % r == 0:
            return r
    return n


@jax.jit
def kernel(h, batch, node_mask, paratope_prob, sasa_prior, W):
    n, d = h.shape
    r = _pick_block(n)
    nb = n // r

    b3 = batch.astype(jnp.int32).reshape(nb, 1, r)
    mk3 = node_mask.astype(jnp.float32).reshape(nb, 1, r)
    pp3 = paratope_prob.astype(jnp.float32).reshape(nb, 1, r)
    ss3 = sasa_prior.astype(jnp.float32).reshape(nb, 1, r)
    w0 = W.astype(jnp.float32).reshape(d, 1)

    row_spec = pl.BlockSpec((1, 1, r), lambda p, i: (i, 0, 0))
    out = pl.pallas_call(
        _body,
        grid=(2, nb),
        in_specs=[
            pl.BlockSpec((r, d), lambda p, i: (i, 0)),
            row_spec, row_spec, row_spec, row_spec,
            pl.BlockSpec((d, 1), lambda p, i: (0, 0)),
        ],
        out_specs=pl.BlockSpec((_NSEG, d), lambda p, i: (0, 0)),
        out_shape=jax.ShapeDtypeStruct((_NSEG, d), jnp.float32),
        scratch_shapes=[
            pltpu.VMEM((1, _NSEG), jnp.float32),
            pltpu.VMEM((1, _NSEG), jnp.float32),
        ],
    )(h.astype(jnp.float32), b3, mk3, pp3, ss3, w0)
    return out


# speedup vs baseline: 8.9735x; 1.6621x over previous
import jax
import jax.numpy as jnp
from jax.experimental import pallas as pl
from jax.experimental.pallas import tpu as pltpu

_BETA = 1.0
_GAMMA = 0.5
_NSEG = 512
_NEG = -1e30


def _body(h_ref, b_ref, mk_ref, pp_ref, ss_ref, w_ref, out_ref,
          m_ref, s_ref, acc_ref):
    i = pl.program_id(0)
    nb = pl.num_programs(0)
    r = h_ref.shape[0]

    h = h_ref[...]
    lcol = jnp.dot(h, w_ref[...], preferred_element_type=jnp.float32)
    lrow = lcol.reshape(1, r)
    batch = b_ref[0]
    mask = mk_ref[0] > 0.5
    ml = jnp.where(mask, lrow + _BETA * pp_ref[0] + _GAMMA * ss_ref[0], _NEG)

    iot = jax.lax.broadcasted_iota(jnp.int32, (_NSEG, r), 0)
    oh = iot == batch
    ohf = oh.astype(jnp.float32)

    @pl.when(i == 0)
    def _init():
        m_ref[...] = jnp.full((1, _NSEG), _NEG, jnp.float32)
        s_ref[...] = jnp.zeros((1, _NSEG), jnp.float32)
        acc_ref[...] = jnp.zeros((_NSEG, h_ref.shape[1]), jnp.float32)

    bmax = jnp.max(jnp.where(oh, ml, _NEG), axis=1).reshape(1, _NSEG)
    m_old = m_ref[...]
    m_new = jnp.maximum(m_old, bmax)
    alpha = jnp.exp(m_old - m_new)
    rowm = jnp.dot(m_new, ohf, preferred_element_type=jnp.float32)
    e = jnp.where(mask, jnp.exp(ml - rowm), 0.0)
    bsum = jnp.sum(jnp.where(oh, e, 0.0), axis=1).reshape(1, _NSEG)
    s_ref[...] = s_ref[...] * alpha + bsum
    m_ref[...] = m_new

    wh = (e.reshape(r, 1) * h).astype(jnp.bfloat16)
    acc_ref[...] = (acc_ref[...] * alpha.reshape(_NSEG, 1)
                    + jnp.dot(oh.astype(jnp.bfloat16), wh,
                              preferred_element_type=jnp.float32))

    @pl.when(i == nb - 1)
    def _finish():
        d = s_ref[...].reshape(_NSEG, 1)
        out_ref[...] = acc_ref[...] / jnp.where(d > 0, d, 1.0)


def _pick_block(n):
    for r in range(2048, 7, -8):
        if n % r == 0:
            return r
    return n


@jax.jit
def kernel(h, batch, node_mask, paratope_prob, sasa_prior, W):
    n, d = h.shape
    r = _pick_block(n)
    nb = n // r

    b3 = batch.astype(jnp.int32).reshape(nb, 1, r)
    mk3 = node_mask.astype(jnp.float32).reshape(nb, 1, r)
    pp3 = paratope_prob.astype(jnp.float32).reshape(nb, 1, r)
    ss3 = sasa_prior.astype(jnp.float32).reshape(nb, 1, r)
    w0 = W.astype(jnp.float32).reshape(d, 1)

    row_spec = pl.BlockSpec((1, 1, r), lambda i: (i, 0, 0))
    out = pl.pallas_call(
        _body,
        grid=(nb,),
        in_specs=[
            pl.BlockSpec((r, d), lambda i: (i, 0)),
            row_spec, row_spec, row_spec, row_spec,
            pl.BlockSpec((d, 1), lambda i: (0, 0)),
        ],
        out_specs=pl.BlockSpec((_NSEG, d), lambda i: (0, 0)),
        out_shape=jax.ShapeDtypeStruct((_NSEG, d), jnp.float32),
        scratch_shapes=[
            pltpu.VMEM((1, _NSEG), jnp.float32),
            pltpu.VMEM((1, _NSEG), jnp.float32),
            pltpu.VMEM((_NSEG, d), jnp.float32),
        ],
    )(h.astype(jnp.float32), b3, mk3, pp3, ss3, w0)
    return out
